# Initial kernel scaffold; baseline (speedup 1.0000x reference)
import jax
import jax.numpy as jnp
from jax.experimental import pallas as pl


def kernel(x, edge_index, W1, b1, W2, b2):
    raise NotImplementedError("write your pallas kernel here")


# speedup vs baseline: 45.3633x; 45.3633x over previous
import jax
import jax.numpy as jnp
from jax import lax
from jax.experimental import pallas as pl
from jax.experimental.pallas import tpu as pltpu
from jax.experimental.pallas import tpu_sc as plsc

N = 10000
E = 320000
D = 128
NC, NS, L = 2, 16, 16
NW = NC * NS
EW = E // NW
G = 100
NG = EW // G
NP = 10240
ROWS_W = NP // NS
ZR = 128

_mesh = plsc.VectorSubcoreMesh(core_axis_name="c", subcore_axis_name="s")
_sc_params = pltpu.CompilerParams(needs_layout_passes=False)


def _zero_vmem_1d(ref, n):
    def body(i, _):
        ref[pl.ds(i * L, L)] = jnp.zeros((L,), jnp.float32)
        return 0
    lax.fori_loop(0, n // L, body, 0)


def _deg_body(dst_hbm, out_hbm, dst_v, acc_v):
    c = lax.axis_index("c")
    s = lax.axis_index("s")
    w = c * NS + s
    pltpu.sync_copy(dst_hbm.at[pl.ds(w * EW, EW)], dst_v)
    _zero_vmem_1d(acc_v, N)
    ones = jnp.ones((L,), jnp.float32)

    def body(i, _):
        idx = dst_v[pl.ds(i * L, L)]
        plsc.addupdate_scatter(acc_v, [idx], ones)
        return 0
    lax.fori_loop(0, EW // L, body, 0)
    pltpu.sync_copy(acc_v, out_hbm.at[w, 0])


_deg_kernel = pl.kernel(
    _deg_body,
    out_type=jax.ShapeDtypeStruct((NW, 1, N), jnp.float32),
    mesh=_mesh,
    compiler_params=_sc_params,
    scratch_types=[
        pltpu.VMEM((EW,), jnp.int32),
        pltpu.VMEM((N,), jnp.float32),
    ],
)


def _edge128_body(y_hbm, src_hbm, dst_hbm, out_hbm,
                  src_v, dst_v, b0, b1, acc_sh, g0, g1, s0, s1):
    c = lax.axis_index("c")
    s = lax.axis_index("s")
    w = c * NS + s
    bufs = [b0, b1]
    gsem = [g0, g1]
    ssem = [s0, s1]

    def zb(i, _):
        for k in range(D // L):
            b0[i, pl.ds(k * L, L)] = jnp.zeros((L,), jnp.float32)
        return 0
    lax.fori_loop(0, G, zb, 0)
    base = s * ROWS_W
    for j in range(6):
        pltpu.sync_copy(b0.at[pl.ds(0, 96)], acc_sh.at[pl.ds(base + j * 96, 96)])
    pltpu.sync_copy(b0.at[pl.ds(0, 64)], acc_sh.at[pl.ds(base + 576, 64)])
    plsc.subcore_barrier()

    def issue_gather(g, b):
        pltpu.async_copy(y_hbm.at[src_v.at[g]], bufs[b], gsem[b])

    def wait_gather(b):
        pltpu.make_async_copy(y_hbm.at[src_v.at[0]], bufs[b], gsem[b]).wait()

    def issue_scatter(g, b):
        pltpu.async_copy(bufs[b], acc_sh.at[dst_v.at[g]], ssem[b], add=True)

    def wait_scatter(b):
        pltpu.make_async_copy(bufs[b], acc_sh.at[dst_v.at[0]], ssem[b]).wait()

    NGH = NG // 2
    for h in range(2):
        pltpu.sync_copy(src_hbm.at[w, h], src_v)
        pltpu.sync_copy(dst_hbm.at[w, h], dst_v)

        issue_gather(0, 0)

        def loop_body(i, _):
            for b in range(2):
                g = i * 2 + b

                @pl.when(g + 1 < NGH)
                def _():
                    @pl.when(g >= 1)
                    def _():
                        wait_scatter((b + 1) % 2)
                    issue_gather(g + 1, (b + 1) % 2)
                wait_gather(b)
                issue_scatter(g, b)
            return 0
        lax.fori_loop(0, NGH // 2, loop_body, 0)
        wait_scatter(0)
        wait_scatter(1)

    plsc.subcore_barrier()
    pltpu.sync_copy(acc_sh.at[pl.ds(s * ROWS_W, ROWS_W)],
                    out_hbm.at[c, pl.ds(s * ROWS_W, ROWS_W)])


_edge128_kernel = pl.kernel(
    _edge128_body,
    out_type=jax.ShapeDtypeStruct((NC, NP, D), jnp.float32),
    mesh=_mesh,
    compiler_params=_sc_params,
    scratch_types=[
        pltpu.VMEM((NG // 2, G), jnp.int32),
        pltpu.VMEM((NG // 2, G), jnp.int32),
        pltpu.VMEM((G, D), jnp.float32),
        pltpu.VMEM((G, D), jnp.float32),
        pltpu.VMEM_SHARED((NP, D), jnp.float32),
        pltpu.SemaphoreType.DMA,
        pltpu.SemaphoreType.DMA,
        pltpu.SemaphoreType.DMA,
        pltpu.SemaphoreType.DMA,
    ],
)


def _edge1_body(src_hbm, dst_hbm, t_hbm, out_hbm, src_v, dst_v, t_v, acc_v):
    c = lax.axis_index("c")
    s = lax.axis_index("s")
    w = c * NS + s
    pltpu.sync_copy(src_hbm.at[pl.ds(w * EW, EW)], src_v)
    pltpu.sync_copy(dst_hbm.at[pl.ds(w * EW, EW)], dst_v)
    pltpu.sync_copy(t_hbm, t_v)
    _zero_vmem_1d(acc_v, N)

    def body(i, _):
        si = src_v[pl.ds(i * L, L)]
        di = dst_v[pl.ds(i * L, L)]
        v = plsc.load_gather(t_v, [si])
        plsc.addupdate_scatter(acc_v, [di], v)
        return 0
    lax.fori_loop(0, EW // L, body, 0)
    pltpu.sync_copy(acc_v, out_hbm.at[w, 0])


_edge1_kernel = pl.kernel(
    _edge1_body,
    out_type=jax.ShapeDtypeStruct((NW, 1, N), jnp.float32),
    mesh=_mesh,
    compiler_params=_sc_params,
    scratch_types=[
        pltpu.VMEM((EW,), jnp.int32),
        pltpu.VMEM((EW,), jnp.int32),
        pltpu.VMEM((N,), jnp.float32),
        pltpu.VMEM((N,), jnp.float32),
    ],
)


_RB = 1000


def _mm1_body(x_ref, w_ref, cnt_ref, y_ref, dinv_ref):
    deg = jnp.sum(cnt_ref[...], axis=1, keepdims=True) + 1.0
    dinv = lax.rsqrt(deg)
    xw = jnp.dot(x_ref[...], w_ref[...], preferred_element_type=jnp.float32)
    y_ref[...] = xw * dinv
    dinv_ref[...] = dinv


def _mm1(x, W1, cntT):
    return pl.pallas_call(
        _mm1_body,
        grid=(N // _RB,),
        in_specs=[
            pl.BlockSpec((_RB, D), lambda i: (i, 0)),
            pl.BlockSpec((D, D), lambda i: (0, 0)),
            pl.BlockSpec((_RB, NW), lambda i: (i, 0)),
        ],
        out_specs=[
            pl.BlockSpec((_RB, D), lambda i: (i, 0)),
            pl.BlockSpec((_RB, 1), lambda i: (i, 0)),
        ],
        out_shape=[
            jax.ShapeDtypeStruct((N, D), jnp.float32),
            jax.ShapeDtypeStruct((N, 1), jnp.float32),
        ],
    )(x, W1, cntT)


def _mid_body(a0_ref, a1_ref, y_ref, dinv_ref, b1_ref, w2_ref, t_ref):
    dinv = dinv_ref[...]
    h = dinv * (a0_ref[...] + a1_ref[...] + y_ref[...]) + b1_ref[...]
    h = jnp.maximum(h, 0.0)
    t_ref[...] = dinv * jnp.sum(h * w2_ref[...], axis=1, keepdims=True)


def _mid(a0, a1, y, dinv, b1r, w2r):
    return pl.pallas_call(
        _mid_body,
        grid=(N // _RB,),
        in_specs=[
            pl.BlockSpec((_RB, D), lambda i: (i, 0)),
            pl.BlockSpec((_RB, D), lambda i: (i, 0)),
            pl.BlockSpec((_RB, D), lambda i: (i, 0)),
            pl.BlockSpec((_RB, 1), lambda i: (i, 0)),
            pl.BlockSpec((1, D), lambda i: (0, 0)),
            pl.BlockSpec((1, D), lambda i: (0, 0)),
        ],
        out_specs=pl.BlockSpec((_RB, 1), lambda i: (i, 0)),
        out_shape=jax.ShapeDtypeStruct((N, 1), jnp.float32),
    )(a0, a1, y, dinv, b1r, w2r)


def _fin_body(p_ref, t_ref, dinv_ref, b2_ref, o_ref):
    acc2 = jnp.sum(p_ref[...], axis=1, keepdims=True)
    o_ref[...] = dinv_ref[...] * (acc2 + t_ref[...]) + b2_ref[...]


def _fin(pT, t, dinv, b2r):
    return pl.pallas_call(
        _fin_body,
        grid=(N // _RB,),
        in_specs=[
            pl.BlockSpec((_RB, NW), lambda i: (i, 0)),
            pl.BlockSpec((_RB, 1), lambda i: (i, 0)),
            pl.BlockSpec((_RB, 1), lambda i: (i, 0)),
            pl.BlockSpec((1, 1), lambda i: (0, 0)),
        ],
        out_specs=pl.BlockSpec((_RB, 1), lambda i: (i, 0)),
        out_shape=jax.ShapeDtypeStruct((N, 1), jnp.float32),
    )(pT, t, dinv, b2r)


def kernel(x, edge_index, W1, b1, W2, b2):
    ei = edge_index.astype(jnp.int32)
    src, dst = ei[0], ei[1]
    src2 = src.reshape(NW, 2, NG // 2, G)
    dst2 = dst.reshape(NW, 2, NG // 2, G)

    cnt = _deg_kernel(dst).reshape(NW, N)
    y, dinv = _mm1(x, W1, cnt.T)
    acc = _edge128_kernel(y, src2, dst2)
    t = _mid(acc[0], acc[1], y, dinv,
             b1.reshape(1, D), W2.reshape(1, D))
    p = _edge1_kernel(src, dst, t.reshape(N)).reshape(NW, N)
    out = _fin(p.T, t, dinv, b2.reshape(1, 1))
    return out.reshape(N)


# speedup vs baseline: 55.0105x; 1.2127x over previous
import jax
import jax.numpy as jnp
from jax import lax
from jax.experimental import pallas as pl
from jax.experimental.pallas import tpu as pltpu
from jax.experimental.pallas import tpu_sc as plsc

N = 10000
E = 320000
D = 128
NC, NS, L = 2, 16, 16
NW = NC * NS
EW = E // NW
G = 100
NG = EW // G
NP = 10240
ROWS_W = NP // NS
ZR = 128

_mesh = plsc.VectorSubcoreMesh(core_axis_name="c", subcore_axis_name="s")
_sc_params = pltpu.CompilerParams(needs_layout_passes=False)


def _zero_vmem_1d(ref, n):
    def body(i, _):
        ref[pl.ds(i * L, L)] = jnp.zeros((L,), jnp.float32)
        return 0
    lax.fori_loop(0, n // L, body, 0)


def _deg_body(ei_hbm, out_hbm, dst_v, acc_v):
    c = lax.axis_index("c")
    s = lax.axis_index("s")
    w = c * NS + s
    pltpu.sync_copy(ei_hbm.at[1, w, 0], dst_v)
    _zero_vmem_1d(acc_v, N)
    ones = jnp.ones((L,), jnp.float32)

    def body(i, _):
        idx = dst_v[pl.ds(i * L, L)]
        plsc.addupdate_scatter(acc_v, [idx], ones)
        return 0
    lax.fori_loop(0, EW // L, body, 0)
    pltpu.sync_copy(acc_v, out_hbm.at[w, 0])


_deg_kernel = pl.kernel(
    _deg_body,
    out_type=jax.ShapeDtypeStruct((NW, 1, N), jnp.float32),
    mesh=_mesh,
    compiler_params=_sc_params,
    scratch_types=[
        pltpu.VMEM((EW,), jnp.int32),
        pltpu.VMEM((N,), jnp.float32),
    ],
)


def _edge128_body(y_hbm, ei_hbm, out_hbm,
                  src_v, dst_v, b0, b1, acc_sh, g0, g1, s0, s1):
    c = lax.axis_index("c")
    s = lax.axis_index("s")
    w = c * NS + s
    bufs = [b0, b1]
    gsem = [g0, g1]
    ssem = [s0, s1]

    def zb(i, _):
        for k in range(D // L):
            b0[i, pl.ds(k * L, L)] = jnp.zeros((L,), jnp.float32)
        return 0
    lax.fori_loop(0, G, zb, 0)
    base = s * ROWS_W
    for j in range(6):
        pltpu.sync_copy(b0.at[pl.ds(0, 96)], acc_sh.at[pl.ds(base + j * 96, 96)])
    pltpu.sync_copy(b0.at[pl.ds(0, 64)], acc_sh.at[pl.ds(base + 576, 64)])
    plsc.subcore_barrier()

    def issue_gather(g, b):
        pltpu.async_copy(y_hbm.at[src_v.at[g]], bufs[b], gsem[b])

    def wait_gather(b):
        pltpu.make_async_copy(y_hbm.at[src_v.at[0]], bufs[b], gsem[b]).wait()

    def issue_scatter(g, b):
        pltpu.async_copy(bufs[b], acc_sh.at[dst_v.at[g]], ssem[b], add=True)

    def wait_scatter(b):
        pltpu.make_async_copy(bufs[b], acc_sh.at[dst_v.at[0]], ssem[b]).wait()

    NGH = NG // 2
    for h in range(2):
        pltpu.sync_copy(ei_hbm.at[0, w, h], src_v)
        pltpu.sync_copy(ei_hbm.at[1, w, h], dst_v)

        issue_gather(0, 0)

        def loop_body(i, _):
            for b in range(2):
                g = i * 2 + b

                @pl.when(g + 1 < NGH)
                def _():
                    @pl.when(g >= 1)
                    def _():
                        wait_scatter((b + 1) % 2)
                    issue_gather(g + 1, (b + 1) % 2)
                wait_gather(b)
                issue_scatter(g, b)
            return 0
        lax.fori_loop(0, NGH // 2, loop_body, 0)
        wait_scatter(0)
        wait_scatter(1)

    plsc.subcore_barrier()
    pltpu.sync_copy(acc_sh.at[pl.ds(s * ROWS_W, ROWS_W)],
                    out_hbm.at[c, pl.ds(s * ROWS_W, ROWS_W)])


_edge128_kernel = pl.kernel(
    _edge128_body,
    out_type=jax.ShapeDtypeStruct((NC, NP, D), jnp.float32),
    mesh=_mesh,
    compiler_params=_sc_params,
    scratch_types=[
        pltpu.VMEM((NG // 2, G), jnp.int32),
        pltpu.VMEM((NG // 2, G), jnp.int32),
        pltpu.VMEM((G, D), jnp.float32),
        pltpu.VMEM((G, D), jnp.float32),
        pltpu.VMEM_SHARED((NP, D), jnp.float32),
        pltpu.SemaphoreType.DMA,
        pltpu.SemaphoreType.DMA,
        pltpu.SemaphoreType.DMA,
        pltpu.SemaphoreType.DMA,
    ],
)


def _edge1_body(ei_hbm, t_hbm, out_hbm, src_v, dst_v, t_v, acc_v):
    c = lax.axis_index("c")
    s = lax.axis_index("s")
    w = c * NS + s
    pltpu.sync_copy(ei_hbm.at[0, w, 0], src_v)
    pltpu.sync_copy(ei_hbm.at[1, w, 0], dst_v)
    pltpu.sync_copy(t_hbm.at[0], t_v)
    _zero_vmem_1d(acc_v, N)

    def body(i, _):
        si = src_v[pl.ds(i * L, L)]
        di = dst_v[pl.ds(i * L, L)]
        v = plsc.load_gather(t_v, [si])
        plsc.addupdate_scatter(acc_v, [di], v)
        return 0
    lax.fori_loop(0, EW // L, body, 0)
    pltpu.sync_copy(acc_v, out_hbm.at[w, 0])


_edge1_kernel = pl.kernel(
    _edge1_body,
    out_type=jax.ShapeDtypeStruct((NW, 1, N), jnp.float32),
    mesh=_mesh,
    compiler_params=_sc_params,
    scratch_types=[
        pltpu.VMEM((EW,), jnp.int32),
        pltpu.VMEM((EW,), jnp.int32),
        pltpu.VMEM((N,), jnp.float32),
        pltpu.VMEM((N,), jnp.float32),
    ],
)


def _mm1_body(x_ref, w_ref, cnt_ref, y_ref, dinv_ref):
    deg = jnp.sum(cnt_ref[...], axis=1, keepdims=True) + 1.0
    dinv = lax.rsqrt(deg)
    xw = jnp.dot(x_ref[...], w_ref[...], preferred_element_type=jnp.float32)
    y_ref[...] = xw * dinv
    dinv_ref[...] = dinv


def _mm1(x, W1, cntT):
    return pl.pallas_call(
        _mm1_body,
        out_shape=[
            jax.ShapeDtypeStruct((N, D), jnp.float32),
            jax.ShapeDtypeStruct((N, 1), jnp.float32),
        ],
    )(x, W1, cntT)


def _mid_body(acc_ref, y_ref, dinv_ref, b1_ref, w2_ref, t_ref):
    dinv = dinv_ref[...]
    h = dinv * (acc_ref[0, :N] + acc_ref[1, :N] + y_ref[...]) + b1_ref[...]
    h = jnp.maximum(h, 0.0) * dinv
    t_ref[...] = lax.dot_general(
        w2_ref[...], h, (((1,), (1,)), ((), ())),
        preferred_element_type=jnp.float32)


def _mid(acc, y, dinv, b1r, w2r):
    return pl.pallas_call(
        _mid_body,
        out_shape=jax.ShapeDtypeStruct((1, N), jnp.float32),
    )(acc, y, dinv, b1r, w2r)


def _fin_body(p_ref, t_ref, cnt_ref, b2_ref, o_ref):
    acc2 = jnp.sum(p_ref[...], axis=0, keepdims=True)
    deg = jnp.sum(cnt_ref[...], axis=0, keepdims=True) + 1.0
    dinv = lax.rsqrt(deg)
    o_ref[...] = dinv * (acc2 + t_ref[...]) + b2_ref[...]


def _fin(p, t, cnt, b2r):
    return pl.pallas_call(
        _fin_body,
        out_shape=jax.ShapeDtypeStruct((1, N), jnp.float32),
    )(p, t, cnt, b2r)


def kernel(x, edge_index, W1, b1, W2, b2):
    ei = edge_index.astype(jnp.int32)
    ei4 = ei.reshape(2, NW, 2, NG // 2, G)

    ei3 = ei.reshape(2, NW, 1, EW)
    cnt = _deg_kernel(ei3).reshape(NW, N)
    y, dinv = _mm1(x, W1, cnt.T)
    acc = _edge128_kernel(y, ei4)
    t = _mid(acc, y, dinv,
             b1.reshape(1, D), W2.reshape(1, D))
    p = _edge1_kernel(ei3, t).reshape(NW, N)
    out = _fin(p, t, cnt, b2.reshape(1, 1))
    return out.reshape(N)
